# baseline (speedup 1.0000x reference)
import functools

import jax
import jax.numpy as jnp
from jax import lax
from jax.experimental import pallas as pl
from jax.experimental.pallas import tpu as pltpu
from jax.experimental.pallas import tpu_sc as plsc

HIDDEN = 64
BATCH = 16384
NSYM = 1000000
NW = 32
B_PER_W = BATCH // NW
RB = 15
NOWN = ((NSYM - 1) >> RB) + 1
CW = 512
FULL_CH = (1 << RB) // CW
LAST_FULL = (NSYM & ((1 << RB) - 1)) // CW
TAILB = NSYM - ((NSYM >> RB) << RB) - LAST_FULL * CW
TAIL0 = NSYM - TAILB
BINW = NW * B_PER_W
CNTW = 48


def _tanh(v):
  return 1.0 - 2.0 / (jnp.exp(2.0 * v) + 1.0)


def _mesh():
  return plsc.VectorSubcoreMesh(core_axis_name="c", subcore_axis_name="s")


def _sc_bin(old_token, new_token):
  @functools.partial(
      pl.kernel,
      out_type=[jax.ShapeDtypeStruct((NW * CNTW,), jnp.int32),
                jax.ShapeDtypeStruct((NW * CNTW,), jnp.int32),
                jax.ShapeDtypeStruct((NW * BINW,), jnp.int32),
                jax.ShapeDtypeStruct((NW * BINW,), jnp.int32),
                jax.ShapeDtypeStruct((NW * BINW,), jnp.int32),
                jax.ShapeDtypeStruct((NW * BINW,), jnp.int32)],
      mesh=_mesh(),
      scratch_types=[
          pltpu.VMEM((B_PER_W,), jnp.int32),
          pltpu.VMEM((B_PER_W,), jnp.int32),
          pltpu.VMEM((CNTW,), jnp.int32),
          pltpu.VMEM((CNTW,), jnp.int32),
          pltpu.VMEM((BINW,), jnp.int32),
          pltpu.VMEM((BINW,), jnp.int32),
          pltpu.VMEM((BINW,), jnp.int32),
          pltpu.VMEM((BINW,), jnp.int32),
      ],
      compiler_params=pltpu.CompilerParams(use_tc_tiling_on_sc=True, needs_layout_passes=False),
  )
  def k(old_hbm, new_hbm, ocnt_hbm, ncnt_hbm, obt_hbm, obs_hbm, nbt_hbm,
        nbs_hbm, otok, ntok, ocv, ncv, obt, obs, nbt, nbs):
    wid = lax.axis_index("s") * 2 + lax.axis_index("c")
    base = wid * B_PER_W
    pltpu.sync_copy(old_hbm.at[pl.ds(base, B_PER_W)], otok)
    pltpu.sync_copy(new_hbm.at[pl.ds(base, B_PER_W)], ntok)
    iota = lax.iota(jnp.int32, 16)
    lane0 = iota == 0

    for tok_ref, bt, bs, cv in ((otok, obt, obs, ocv), (ntok, nbt, nbs, ncv)):
      def per_owner(v, _, tok_ref=tok_ref, bt=bt, bs=bs, cv=cv):
        def g(gi, off, tok_ref=tok_ref, bt=bt, bs=bs, v=v):
          tok = tok_ref[pl.ds(gi * 16, 16)]
          m = lax.shift_right_logical(tok, RB) == v
          slots = base + gi * 16 + iota
          key = jnp.where(m, iota, iota + 16)
          _, ctok = plsc.sort_key_val(key, tok)
          _, cslt = plsc.sort_key_val(key, slots)
          doff = v * B_PER_W + off
          bt[pl.ds(doff, 16)] = ctok
          bs[pl.ds(doff, 16)] = cslt
          return off + plsc.all_reduce_population_count(m)[0]
        cnt = lax.fori_loop(0, B_PER_W // 16, g, 0)
        plsc.store_scatter(cv, [jnp.full((16,), v, jnp.int32)],
                           jnp.full((16,), cnt, jnp.int32), mask=lane0)
        return 0
      lax.fori_loop(0, NOWN, per_owner, 0)

    pltpu.sync_copy(ocv, ocnt_hbm.at[pl.ds(wid * CNTW, CNTW)])
    pltpu.sync_copy(ncv, ncnt_hbm.at[pl.ds(wid * CNTW, CNTW)])
    pltpu.sync_copy(obt, obt_hbm.at[pl.ds(wid * BINW, BINW)])
    pltpu.sync_copy(obs, obs_hbm.at[pl.ds(wid * BINW, BINW)])
    pltpu.sync_copy(nbt, nbt_hbm.at[pl.ds(wid * BINW, BINW)])
    pltpu.sync_copy(nbs, nbs_hbm.at[pl.ds(wid * BINW, BINW)])

  return k(old_token, new_token)


def _sc_stream(embT, emb_tail, ocnt, ncnt, obt, obs, nbt, nbs):
  @functools.partial(
      pl.kernel,
      out_type=[jax.ShapeDtypeStruct((BATCH, HIDDEN), jnp.float32),
                jax.ShapeDtypeStruct((BATCH, HIDDEN), jnp.float32)],
      mesh=_mesh(),
      scratch_types=[
          pltpu.VMEM((HIDDEN, CW), jnp.float32),
          pltpu.VMEM((NW * CNTW,), jnp.int32),
          pltpu.VMEM((NW * CNTW,), jnp.int32),
          pltpu.VMEM((BINW,), jnp.int32),
          pltpu.VMEM((BINW,), jnp.int32),
          pltpu.VMEM((BINW,), jnp.int32),
          pltpu.VMEM((BINW,), jnp.int32),
          pltpu.VMEM((B_PER_W + 16,), jnp.int32),
          pltpu.VMEM((B_PER_W + 16,), jnp.int32),
          pltpu.VMEM((16, HIDDEN), jnp.float32),
          pltpu.VMEM((HIDDEN, 128), jnp.float32),
          pltpu.SemaphoreType.DMA,
          pltpu.SemaphoreType.DMA,
      ],
      compiler_params=pltpu.CompilerParams(use_tc_tiling_on_sc=True, needs_layout_passes=False),
  )
  def k(table, tail_hbm, ocnt_hbm, ncnt_hbm, obt_hbm, obs_hbm, nbt_hbm,
        nbs_hbm, x_hbm, y_hbm, chunkb, ocv, ncv, lot, los, lnt, lns,
        stok, sslt, rowst, tailb, sem, rsem):
    v = lax.axis_index("s") * 2 + lax.axis_index("c")
    pltpu.sync_copy(ocnt_hbm, ocv)
    pltpu.sync_copy(ncnt_hbm, ncv)
    for src, dstr in ((obt_hbm, lot), (obs_hbm, los),
                      (nbt_hbm, lnt), (nbs_hbm, lns)):
      for w in range(NW):
        pltpu.async_copy(
            src.at[pl.ds(w * BINW + v * B_PER_W, B_PER_W)],
            dstr.at[pl.ds(w * B_PER_W, B_PER_W)], sem)
      pltpu.make_async_copy(src.at[pl.ds(0, BINW)], dstr, sem).wait()
    iota = lax.iota(jnp.int32, 16)
    tbase = v << RB

    def do_chunk(buf, coff, clen):
      for cv, lt, ls, dst in ((ocv, lot, los, x_hbm), (ncv, lnt, lns, y_hbm)):
        def scan_w(w, _, cv=cv, lt=lt, ls=ls, dst=dst):
          wcnt = cv[pl.ds(w * CNTW + v, 16)][0]

          def g(gi, off2, w=w, lt=lt, ls=ls, wcnt=wcnt):
            tok = lt[pl.ds(w * B_PER_W + gi * 16, 16)]
            slt = ls[pl.ds(w * B_PER_W + gi * 16, 16)]
            valid = gi * 16 + iota < wcnt
            m = jnp.logical_and(
                jnp.logical_and(tok >= coff, tok < coff + clen), valid)
            key = jnp.where(m, iota, iota + 16)
            _, ctok = plsc.sort_key_val(key, tok - coff)
            _, cslt = plsc.sort_key_val(key, slt)
            stok[pl.ds(off2, 16)] = ctok
            sslt[pl.ds(off2, 16)] = cslt
            return off2 + plsc.all_reduce_population_count(m)[0]
          hcnt = lax.fori_loop(0, (wcnt + 15) >> 4, g, 0)

          def extract(hg, _, dst=dst, hcnt=hcnt):
            hb = hg * 16
            tl = stok[pl.ds(hb, 16)]
            sl = sslt[pl.ds(hb, 16)]
            valid = hb + iota < hcnt
            tl = jnp.where(valid, tl, 0)
            for c in range(HIDDEN):
              vals = plsc.load_gather(
                  buf, [jnp.full((16,), c, jnp.int32), tl])
              plsc.store_scatter(rowst,
                                 [iota, jnp.full((16,), c, jnp.int32)],
                                 _tanh(vals))
            for li in range(16):
              @pl.when(hb + li < hcnt)
              def _(li=li, sl=sl, dst=dst):
                pltpu.sync_copy(rowst.at[pl.ds(li, 1), :],
                                dst.at[pl.ds(sl[li], 1), :])
            return 0
          lax.fori_loop(0, (hcnt + 15) >> 4, extract, 0)
          return 0
        lax.fori_loop(0, NW, scan_w, 0)

    nch = jnp.where(v < NOWN - 1, FULL_CH,
                    jnp.where(v == NOWN - 1, LAST_FULL, 0))

    def chunk_loop(c, _):
      coff = tbase + c * CW
      pltpu.sync_copy(
          table.at[:, pl.ds(pl.multiple_of(coff, 128), CW)], chunkb)
      do_chunk(chunkb, coff, CW)
      return 0
    lax.fori_loop(0, nch, chunk_loop, 0)

    @pl.when(v == NOWN - 1)
    def _():
      pltpu.sync_copy(tail_hbm, tailb)
      do_chunk(tailb, TAIL0, TAILB)

  return k(embT, emb_tail, ocnt, ncnt, obt, obs, nbt, nbs)


def _combine_body(h_ref, x_ref, y_ref, pos_ref, gamma_ref, lin_ref, g_ref,
                  b_ref, o_ref):
  pf = pos_ref[...].astype(jnp.float32)
  scale = (g_ref[...] * jnp.power(gamma_ref[...], pf)
           + pf * lin_ref[...] + b_ref[...])
  o_ref[...] = h_ref[...] + (y_ref[...] - x_ref[...]) * scale


def _tc_combine(h, x, y, pos, gamma, lin, g, b):
  BS = 2048
  row = pl.BlockSpec((BS, HIDDEN), lambda i: (i, 0))
  vec = pl.BlockSpec((1, HIDDEN), lambda i: (0, 0))
  return pl.pallas_call(
      _combine_body,
      grid=(BATCH // BS,),
      in_specs=[row, row, row, pl.BlockSpec((BS, 1), lambda i: (i, 0)),
                vec, vec, vec, vec],
      out_specs=row,
      out_shape=jax.ShapeDtypeStruct((BATCH, HIDDEN), jnp.float32),
  )(h, x, y, pos.reshape(BATCH, 1), gamma.reshape(1, HIDDEN),
    lin.reshape(1, HIDDEN), g.reshape(1, HIDDEN), b.reshape(1, HIDDEN))


def kernel(h, old_token, new_token, pos, embedding, gamma, lin, g, b):
  ocnt, ncnt, obt, obs, nbt, nbs = _sc_bin(old_token, new_token)
  emb_tail = jnp.concatenate(
      [embedding.T[:, TAIL0:], jnp.zeros((HIDDEN, 128 - TAILB), jnp.float32)],
      axis=1)
  x, y = _sc_stream(embedding.T, emb_tail, ocnt, ncnt, obt, obs, nbt, nbs)
  return _tc_combine(h, x, y, pos, gamma, lin, g, b)


# speedup vs baseline: 3.9096x; 3.9096x over previous
import functools

import jax
import jax.numpy as jnp
from jax import lax
from jax.experimental import pallas as pl
from jax.experimental.pallas import tpu as pltpu
from jax.experimental.pallas import tpu_sc as plsc

HIDDEN = 64
BATCH = 16384
NSYM = 1000000
NW = 32
B_PER_W = BATCH // NW
RB = 15
NOWN = ((NSYM - 1) >> RB) + 1
CW = 512
FULL_CH = (1 << RB) // CW
LAST_FULL = (NSYM & ((1 << RB) - 1)) // CW
TAILB = NSYM - ((NSYM >> RB) << RB) - LAST_FULL * CW
TAIL0 = NSYM - TAILB
GCHUNK = 32
NGCH = B_PER_W // GCHUNK
TPC = GCHUNK // 8


def _tanh(v):
  return 1.0 - 2.0 / (jnp.exp(2.0 * v) + 1.0)


def _mesh():
  return plsc.VectorSubcoreMesh(core_axis_name="c", subcore_axis_name="s")

_PARAMS = dict(
    compiler_params=pltpu.CompilerParams(use_tc_tiling_on_sc=True,
                                         needs_layout_passes=False))


def _sc_mark(old4, new4):
  @functools.partial(
      pl.kernel,
      out_type=[jax.ShapeDtypeStruct((NSYM,), jnp.int32),
                jax.ShapeDtypeStruct((NSYM,), jnp.int32)],
      mesh=_mesh(),
      scratch_types=[
          pltpu.VMEM((4, 128), jnp.int32),
          pltpu.VMEM((4, 128), jnp.int32),
          pltpu.VMEM((4, 128), jnp.int32),
          pltpu.SemaphoreType.DMA,
      ],
      **_PARAMS,
  )
  def k(old_hbm, new_hbm, ost_hbm, nst_hbm, tko, tkn, slv, sem):
    wid = lax.axis_index("s") * 2 + lax.axis_index("c")
    base = wid * B_PER_W
    pltpu.sync_copy(old_hbm.at[pl.ds(wid * 4, 4)], tko)
    pltpu.sync_copy(new_hbm.at[pl.ds(wid * 4, 4)], tkn)
    iota = lax.iota(jnp.int32, 16)
    for j in range(4):
      for kk in range(8):
        slv[j, pl.ds(kk * 16, 16)] = base + j * 128 + kk * 16 + iota
    for j in range(4):
      pltpu.async_copy(slv.at[j], ost_hbm.at[tko.at[j]], sem)
      pltpu.async_copy(slv.at[j], nst_hbm.at[tkn.at[j]], sem)
    for j in range(4):
      pltpu.make_async_copy(slv.at[j], ost_hbm.at[tko.at[j]], sem).wait()
      pltpu.make_async_copy(slv.at[j], nst_hbm.at[tkn.at[j]], sem).wait()

  return k(old4, new4)


def _sc_tanh_rows(embT, emb_tail, old_token, new_token, ost, nst):
  @functools.partial(
      pl.kernel,
      out_type=jax.ShapeDtypeStruct((NSYM, HIDDEN), jnp.float32),
      mesh=_mesh(),
      scratch_types=[
          pltpu.VMEM((HIDDEN, CW), jnp.float32),
          pltpu.VMEM((HIDDEN, 128), jnp.float32),
          pltpu.VMEM((BATCH,), jnp.int32),
          pltpu.VMEM((BATCH,), jnp.int32),
          pltpu.VMEM((CW,), jnp.int32),
          pltpu.VMEM((CW,), jnp.int32),
          pltpu.VMEM((CW + 16,), jnp.int32),
          pltpu.VMEM((16, HIDDEN), jnp.float32),
      ],
      **_PARAMS,
  )
  def k(table, tail_hbm, old_hbm, new_hbm, ost_hbm, nst_hbm, tt_hbm,
        chunkb, tailb, otok, ntok, osw, nsw, scol, rowst):
    v = lax.axis_index("s") * 2 + lax.axis_index("c")
    pltpu.sync_copy(old_hbm, otok)
    pltpu.sync_copy(new_hbm, ntok)
    iota = lax.iota(jnp.int32, 16)
    tbase = v << RB

    def do_chunk(buf, coff, clen):
      def scan(gi, off):
        gb = gi * 16
        so = osw[pl.ds(gb, 16)]
        sn = nsw[pl.ds(gb, 16)]
        exp_tok = coff + gb + iota
        vo = jnp.logical_and(so >= 0, so < BATCH)
        mo = jnp.logical_and(
            vo, plsc.load_gather(otok, [jnp.where(vo, so, 0)]) == exp_tok)
        vn = jnp.logical_and(sn >= 0, sn < BATCH)
        mn = jnp.logical_and(
            vn, plsc.load_gather(ntok, [jnp.where(vn, sn, 0)]) == exp_tok)
        m = jnp.logical_and(jnp.logical_or(mo, mn), gb + iota < clen)
        key = jnp.where(m, iota, iota + 16)
        _, ccol = plsc.sort_key_val(key, gb + iota)
        scol[pl.ds(off, 16)] = ccol
        return off + plsc.all_reduce_population_count(m)[0]
      hcnt = lax.fori_loop(0, CW // 16, scan, 0)

      def extract(hg, _):
        hb = hg * 16
        valid = hb + iota < hcnt
        cl = jnp.where(valid, scol[pl.ds(hb, 16)], 0)
        for c in range(HIDDEN):
          vals = plsc.load_gather(buf, [jnp.full((16,), c, jnp.int32), cl])
          plsc.store_scatter(rowst, [iota, jnp.full((16,), c, jnp.int32)],
                             _tanh(vals))
        for li in range(16):
          @pl.when(hb + li < hcnt)
          def _(li=li, cl=cl):
            pltpu.sync_copy(rowst.at[pl.ds(li, 1), :],
                            tt_hbm.at[pl.ds(coff + cl[li], 1), :])
        return 0
      lax.fori_loop(0, (hcnt + 15) >> 4, extract, 0)

    nch = jnp.where(v < NOWN - 1, FULL_CH,
                    jnp.where(v == NOWN - 1, LAST_FULL, 0))

    def chunk_loop(c, _):
      coff = tbase + c * CW
      aligned = pl.multiple_of(coff, 128)
      pltpu.sync_copy(table.at[:, pl.ds(aligned, CW)], chunkb)
      pltpu.sync_copy(ost_hbm.at[pl.ds(aligned, CW)], osw)
      pltpu.sync_copy(nst_hbm.at[pl.ds(aligned, CW)], nsw)
      do_chunk(chunkb, coff, CW)
      return 0
    lax.fori_loop(0, nch, chunk_loop, 0)

    @pl.when(v == NOWN - 1)
    def _():
      pltpu.sync_copy(tail_hbm, tailb)
      pltpu.sync_copy(ost_hbm.at[pl.ds(TAIL0, TAILB)],
                      osw.at[pl.ds(0, TAILB)])
      pltpu.sync_copy(nst_hbm.at[pl.ds(TAIL0, TAILB)],
                      nsw.at[pl.ds(0, TAILB)])
      do_chunk(tailb, TAIL0, TAILB)

  return k(embT, emb_tail, old_token, new_token, ost, nst)


def _sc_gather_combine(tt, old_token, new_token, pos, h3, lin, g, b, lg):
  @functools.partial(
      pl.kernel,
      out_type=jax.ShapeDtypeStruct((BATCH // 8, 8, HIDDEN), jnp.float32),
      mesh=_mesh(),
      scratch_types=[
          pltpu.VMEM((B_PER_W + 16,), jnp.int32),
          pltpu.VMEM((B_PER_W + 16,), jnp.int32),
          pltpu.VMEM((B_PER_W + 16,), jnp.int32),
          pltpu.VMEM((HIDDEN,), jnp.float32),
          pltpu.VMEM((HIDDEN,), jnp.float32),
          pltpu.VMEM((HIDDEN,), jnp.float32),
          pltpu.VMEM((HIDDEN,), jnp.float32),
          pltpu.VMEM((GCHUNK, HIDDEN), jnp.float32),
          pltpu.VMEM((GCHUNK, HIDDEN), jnp.float32),
          pltpu.VMEM((TPC, 8, HIDDEN), jnp.float32),
          pltpu.VMEM((TPC, 8, HIDDEN), jnp.float32),
          pltpu.SemaphoreType.DMA,
      ],
      **_PARAMS,
  )
  def k(table, old_hbm, new_hbm, pos_hbm, h_hbm, lin_hbm, g_hbm, b_hbm,
        lg_hbm, out_hbm, oidx, nidx, posv, gv, linv, bv, lgv, xt, yt,
        hb, ob, sem):
    wid = lax.axis_index("s") * 2 + lax.axis_index("c")
    base = wid * B_PER_W
    pltpu.sync_copy(old_hbm.at[pl.ds(base, B_PER_W)],
                    oidx.at[pl.ds(0, B_PER_W)])
    pltpu.sync_copy(new_hbm.at[pl.ds(base, B_PER_W)],
                    nidx.at[pl.ds(0, B_PER_W)])
    pltpu.sync_copy(pos_hbm.at[pl.ds(base, B_PER_W)],
                    posv.at[pl.ds(0, B_PER_W)])
    pltpu.sync_copy(g_hbm, gv)
    pltpu.sync_copy(lin_hbm, linv)
    pltpu.sync_copy(b_hbm, bv)
    pltpu.sync_copy(lg_hbm, lgv)

    gvec = [gv[pl.ds(16 * j, 16)] for j in range(4)]
    linvec = [linv[pl.ds(16 * j, 16)] for j in range(4)]
    bvec = [bv[pl.ds(16 * j, 16)] for j in range(4)]
    lgvec = [lgv[pl.ds(16 * j, 16)] for j in range(4)]

    for c in range(NGCH):
      cb = c * GCHUNK

      def issue(i, _):
        gi = cb + i
        orow = oidx[pl.ds(gi, 16)][0]
        nrow = nidx[pl.ds(gi, 16)][0]
        pltpu.async_copy(table.at[pl.ds(orow, 1), :],
                         xt.at[pl.ds(i, 1), :], sem)
        pltpu.async_copy(table.at[pl.ds(nrow, 1), :],
                         yt.at[pl.ds(i, 1), :], sem)
        return 0
      lax.fori_loop(0, GCHUNK, issue, 0)
      pltpu.sync_copy(h_hbm.at[pl.ds(base // 8 + c * TPC, TPC)], hb)
      pltpu.make_async_copy(table.at[pl.ds(0, GCHUNK), :], xt, sem).wait()
      pltpu.make_async_copy(table.at[pl.ds(0, GCHUNK), :], yt, sem).wait()

      def body(i, _):
        gi = cb + i
        pf = posv[pl.ds(gi, 16)][0].astype(jnp.float32)
        it = lax.shift_right_logical(i, 3)
        is_ = lax.bitwise_and(i, 7)
        for j in range(4):
          sl = pl.ds(16 * j, 16)
          xv = xt[i, sl]
          yv = yt[i, sl]
          hv = hb[it, is_, sl]
          scale = gvec[j] * jnp.exp(pf * lgvec[j]) + pf * linvec[j] + bvec[j]
          ob[it, is_, sl] = hv + (yv - xv) * scale
        return 0
      lax.fori_loop(0, GCHUNK, body, 0)

      pltpu.sync_copy(ob, out_hbm.at[pl.ds(base // 8 + c * TPC, TPC)])

  return k(tt, old_token, new_token, pos, h3, lin, g, b, lg)


def kernel(h, old_token, new_token, pos, embedding, gamma, lin, g, b):
  embT = embedding.T
  emb_tail = jnp.concatenate(
      [embT[:, TAIL0:], jnp.zeros((HIDDEN, 128 - TAILB), jnp.float32)],
      axis=1)
  old4 = old_token.reshape(NW * 4, 128)
  new4 = new_token.reshape(NW * 4, 128)
  ost, nst = _sc_mark(old4, new4)
  tt = _sc_tanh_rows(embT, emb_tail, old_token, new_token, ost, nst)
  h3 = h.reshape(BATCH // 8, 8, HIDDEN)
  lg = jnp.log(gamma)
  out3 = _sc_gather_combine(tt, old_token, new_token, pos, h3,
                            lin, g, b, lg)
  return out3.reshape(BATCH, HIDDEN)


# speedup vs baseline: 5.8563x; 1.4979x over previous
import functools

import jax
import jax.numpy as jnp
from jax import lax
from jax.experimental import pallas as pl
from jax.experimental.pallas import tpu as pltpu
from jax.experimental.pallas import tpu_sc as plsc

HIDDEN = 64
BATCH = 16384
NSYM = 1000000
NW = 32
B_PER_W = BATCH // NW
RB = 15
NOWN = ((NSYM - 1) >> RB) + 1
CW = 512
FULL_CH = (1 << RB) // CW
LAST_FULL = (NSYM & ((1 << RB) - 1)) // CW
TAILB = NSYM - ((NSYM >> RB) << RB) - LAST_FULL * CW
TAIL0 = NSYM - TAILB
GCHUNK = 32
NGCH = B_PER_W // GCHUNK
TPC = GCHUNK // 8


def _tanh(v):
  return 1.0 - 2.0 / (jnp.exp(2.0 * v) + 1.0)


def _mesh():
  return plsc.VectorSubcoreMesh(core_axis_name="c", subcore_axis_name="s")

_PARAMS = dict(
    compiler_params=pltpu.CompilerParams(use_tc_tiling_on_sc=True,
                                         needs_layout_passes=False))


def _sc_mark(old4, new4):
  @functools.partial(
      pl.kernel,
      out_type=jax.ShapeDtypeStruct((NSYM,), jnp.int32),
      mesh=_mesh(),
      scratch_types=[
          pltpu.VMEM((4, 128), jnp.int32),
          pltpu.VMEM((4, 128), jnp.int32),
          pltpu.VMEM((4, 128), jnp.int32),
          pltpu.VMEM((4, 128), jnp.int32),
          pltpu.SemaphoreType.DMA,
      ],
      **_PARAMS,
  )
  def k(old_hbm, new_hbm, st_hbm, tko, tkn, slv, slv2, sem):
    wid = lax.axis_index("s") * 2 + lax.axis_index("c")
    base = wid * B_PER_W
    pltpu.sync_copy(old_hbm.at[pl.ds(wid * 4, 4)], tko)
    pltpu.sync_copy(new_hbm.at[pl.ds(wid * 4, 4)], tkn)
    iota = lax.iota(jnp.int32, 16)
    for j in range(4):
      for kk in range(8):
        s = base + j * 128 + kk * 16 + iota
        slv[j, pl.ds(kk * 16, 16)] = s
        slv2[j, pl.ds(kk * 16, 16)] = s + BATCH
    for j in range(4):
      pltpu.async_copy(slv.at[j], st_hbm.at[tko.at[j]], sem)
      pltpu.async_copy(slv2.at[j], st_hbm.at[tkn.at[j]], sem)
    for j in range(4):
      pltpu.make_async_copy(slv.at[j], st_hbm.at[tko.at[j]], sem).wait()
      pltpu.make_async_copy(slv2.at[j], st_hbm.at[tkn.at[j]], sem).wait()

  return k(old4, new4)


def _sc_tanh_rows(embT, emb_tail, old_token, new_token, st):
  @functools.partial(
      pl.kernel,
      out_type=jax.ShapeDtypeStruct((NSYM, HIDDEN), jnp.float32),
      mesh=_mesh(),
      scratch_types=[
          pltpu.VMEM((HIDDEN, CW), jnp.float32),
          pltpu.VMEM((HIDDEN, CW), jnp.float32),
          pltpu.VMEM((HIDDEN, 128), jnp.float32),
          pltpu.VMEM((2 * BATCH,), jnp.int32),
          pltpu.VMEM((CW,), jnp.int32),
          pltpu.VMEM((CW,), jnp.int32),
          pltpu.VMEM((CW + 16,), jnp.int32),
          pltpu.VMEM((16, HIDDEN), jnp.float32),
          pltpu.SemaphoreType.DMA,
          pltpu.SemaphoreType.DMA,
          pltpu.SemaphoreType.DMA,
      ],
      **_PARAMS,
  )
  def k(table, tail_hbm, old_hbm, new_hbm, st_hbm, tt_hbm,
        chA, chB, tailb, tok_all, swA, swB, scol, rowst, semA, semB, rsem):
    v = lax.axis_index("s") * 2 + lax.axis_index("c")
    pltpu.sync_copy(old_hbm, tok_all.at[pl.ds(0, BATCH)])
    pltpu.sync_copy(new_hbm, tok_all.at[pl.ds(BATCH, BATCH)])
    iota = lax.iota(jnp.int32, 16)
    tbase = v << RB
    nchk = jnp.where(v < NOWN - 1, FULL_CH,
                     jnp.where(v == NOWN - 1, LAST_FULL, 0))
    npair = nchk >> 1

    def fetch(c, ch, sw, sem):
      coff = pl.multiple_of(tbase + c * CW, 128)
      pltpu.async_copy(table.at[:, pl.ds(coff, CW)], ch, sem)
      pltpu.async_copy(st_hbm.at[pl.ds(coff, CW)], sw, sem)

    def waitf(ch, sw, sem):
      pltpu.make_async_copy(table.at[:, pl.ds(0, CW)], ch, sem).wait()
      pltpu.make_async_copy(st_hbm.at[pl.ds(0, CW)], sw, sem).wait()

    def do_chunk(buf, sw, coff, clen):
      def scan(gi, off):
        gb = gi * 16
        s = sw[pl.ds(gb, 16)]
        exp_tok = coff + gb + iota
        vs = jnp.logical_and(s >= 0, s < 2 * BATCH)
        m = jnp.logical_and(
            vs, plsc.load_gather(tok_all, [jnp.where(vs, s, 0)]) == exp_tok)
        m = jnp.logical_and(m, gb + iota < clen)
        cnt = plsc.all_reduce_population_count(m)[0]

        @pl.when(cnt > 0)
        def _():
          key = jnp.where(m, iota, iota + 16)
          _, ccol = plsc.sort_key_val(key, gb + iota)
          scol[pl.ds(off, 16)] = ccol
        return off + cnt
      hcnt = lax.fori_loop(0, CW // 16, scan, 0)

      def extract(hg, _):
        hb = hg * 16
        valid = hb + iota < hcnt
        cl = jnp.where(valid, scol[pl.ds(hb, 16)], 0)
        for c in range(HIDDEN):
          vals = plsc.load_gather(buf, [jnp.full((16,), c, jnp.int32), cl])
          plsc.store_scatter(rowst, [iota, jnp.full((16,), c, jnp.int32)],
                             _tanh(vals))
        for li in range(16):
          @pl.when(hb + li < hcnt)
          def _(li=li, cl=cl):
            pltpu.async_copy(rowst.at[pl.ds(li, 1), :],
                             tt_hbm.at[pl.ds(coff + cl[li], 1), :], rsem)
        for li in range(16):
          @pl.when(hb + li < hcnt)
          def _(li=li):
            pltpu.make_async_copy(rowst.at[pl.ds(li, 1), :],
                                  tt_hbm.at[pl.ds(0, 1), :], rsem).wait()
        return 0
      lax.fori_loop(0, (hcnt + 15) >> 4, extract, 0)

    @pl.when(npair > 0)
    def _():
      fetch(0, chA, swA, semA)

    def pair(c2, _):
      c0 = c2 * 2
      fetch(c0 + 1, chB, swB, semB)
      waitf(chA, swA, semA)
      do_chunk(chA, swA, tbase + c0 * CW, CW)

      @pl.when(c0 + 2 < nchk)
      def _():
        fetch(c0 + 2, chA, swA, semA)
      waitf(chB, swB, semB)
      do_chunk(chB, swB, tbase + (c0 + 1) * CW, CW)
      return 0
    lax.fori_loop(0, npair, pair, 0)

    @pl.when(v == NOWN - 1)
    def _():
      coff = tbase + (LAST_FULL - 1) * CW
      waitf(chA, swA, semA)
      do_chunk(chA, swA, coff, CW)
      pltpu.sync_copy(tail_hbm, tailb)
      pltpu.sync_copy(st_hbm.at[pl.ds(TAIL0, TAILB)],
                      swA.at[pl.ds(0, TAILB)])
      do_chunk(tailb, swA, TAIL0, TAILB)

  return k(embT, emb_tail, old_token, new_token, st)


def _sc_gather_combine(tt, old_token, new_token, pos, h3, lin, g, b, lg):
  @functools.partial(
      pl.kernel,
      out_type=jax.ShapeDtypeStruct((BATCH // 8, 8, HIDDEN), jnp.float32),
      mesh=_mesh(),
      scratch_types=[
          pltpu.VMEM((B_PER_W + 16,), jnp.int32),
          pltpu.VMEM((B_PER_W + 16,), jnp.int32),
          pltpu.VMEM((B_PER_W + 16,), jnp.int32),
          pltpu.VMEM((HIDDEN,), jnp.float32),
          pltpu.VMEM((HIDDEN,), jnp.float32),
          pltpu.VMEM((HIDDEN,), jnp.float32),
          pltpu.VMEM((HIDDEN,), jnp.float32),
          pltpu.VMEM((GCHUNK, HIDDEN), jnp.float32),
          pltpu.VMEM((GCHUNK, HIDDEN), jnp.float32),
          pltpu.VMEM((TPC, 8, HIDDEN), jnp.float32),
          pltpu.VMEM((TPC, 8, HIDDEN), jnp.float32),
          pltpu.SemaphoreType.DMA,
      ],
      **_PARAMS,
  )
  def k(table, old_hbm, new_hbm, pos_hbm, h_hbm, lin_hbm, g_hbm, b_hbm,
        lg_hbm, out_hbm, oidx, nidx, posv, gv, linv, bv, lgv, xt, yt,
        hb, ob, sem):
    wid = lax.axis_index("s") * 2 + lax.axis_index("c")
    base = wid * B_PER_W
    pltpu.sync_copy(old_hbm.at[pl.ds(base, B_PER_W)],
                    oidx.at[pl.ds(0, B_PER_W)])
    pltpu.sync_copy(new_hbm.at[pl.ds(base, B_PER_W)],
                    nidx.at[pl.ds(0, B_PER_W)])
    pltpu.sync_copy(pos_hbm.at[pl.ds(base, B_PER_W)],
                    posv.at[pl.ds(0, B_PER_W)])
    pltpu.sync_copy(g_hbm, gv)
    pltpu.sync_copy(lin_hbm, linv)
    pltpu.sync_copy(b_hbm, bv)
    pltpu.sync_copy(lg_hbm, lgv)

    gvec = [gv[pl.ds(16 * j, 16)] for j in range(4)]
    linvec = [linv[pl.ds(16 * j, 16)] for j in range(4)]
    bvec = [bv[pl.ds(16 * j, 16)] for j in range(4)]
    lgvec = [lgv[pl.ds(16 * j, 16)] for j in range(4)]

    for c in range(NGCH):
      cb = c * GCHUNK

      def issue(i, _):
        gi = cb + i
        orow = oidx[pl.ds(gi, 16)][0]
        nrow = nidx[pl.ds(gi, 16)][0]
        pltpu.async_copy(table.at[pl.ds(orow, 1), :],
                         xt.at[pl.ds(i, 1), :], sem)
        pltpu.async_copy(table.at[pl.ds(nrow, 1), :],
                         yt.at[pl.ds(i, 1), :], sem)
        return 0
      lax.fori_loop(0, GCHUNK, issue, 0)
      pltpu.sync_copy(h_hbm.at[pl.ds(base // 8 + c * TPC, TPC)], hb)
      pltpu.make_async_copy(table.at[pl.ds(0, GCHUNK), :], xt, sem).wait()
      pltpu.make_async_copy(table.at[pl.ds(0, GCHUNK), :], yt, sem).wait()

      def body(i, _):
        gi = cb + i
        pf = posv[pl.ds(gi, 16)][0].astype(jnp.float32)
        it = lax.shift_right_logical(i, 3)
        is_ = lax.bitwise_and(i, 7)
        for j in range(4):
          sl = pl.ds(16 * j, 16)
          xv = xt[i, sl]
          yv = yt[i, sl]
          hv = hb[it, is_, sl]
          scale = gvec[j] * jnp.exp(pf * lgvec[j]) + pf * linvec[j] + bvec[j]
          ob[it, is_, sl] = hv + (yv - xv) * scale
        return 0
      lax.fori_loop(0, GCHUNK, body, 0)

      pltpu.sync_copy(ob, out_hbm.at[pl.ds(base // 8 + c * TPC, TPC)])

  return k(tt, old_token, new_token, pos, h3, lin, g, b, lg)


def kernel(h, old_token, new_token, pos, embedding, gamma, lin, g, b):
  embT = embedding.T
  emb_tail = jnp.concatenate(
      [embT[:, TAIL0:], jnp.zeros((HIDDEN, 128 - TAILB), jnp.float32)],
      axis=1)
  old4 = old_token.reshape(NW * 4, 128)
  new4 = new_token.reshape(NW * 4, 128)
  st = _sc_mark(old4, new4)
  tt = _sc_tanh_rows(embT, emb_tail, old_token, new_token, st)
  h3 = h.reshape(BATCH // 8, 8, HIDDEN)
  lg = jnp.log(gamma)
  out3 = _sc_gather_combine(tt, old_token, new_token, pos, h3,
                            lin, g, b, lg)
  return out3.reshape(BATCH, HIDDEN)


# speedup vs baseline: 5.9413x; 1.0145x over previous
import functools

import jax
import jax.numpy as jnp
from jax import lax
from jax.experimental import pallas as pl
from jax.experimental.pallas import tpu as pltpu
from jax.experimental.pallas import tpu_sc as plsc

HIDDEN = 64
BATCH = 16384
NSYM = 1000000
NW = 32
B_PER_W = BATCH // NW
RB = 15
NOWN = ((NSYM - 1) >> RB) + 1
CW = 512
FULL_CH = (1 << RB) // CW
LAST_FULL = (NSYM & ((1 << RB) - 1)) // CW
TAILB = NSYM - ((NSYM >> RB) << RB) - LAST_FULL * CW
TAIL0 = NSYM - TAILB
GCHUNK = 32
NGCH = B_PER_W // GCHUNK
TPC = GCHUNK // 8


def _tanh(v):
  return 1.0 - 2.0 / (jnp.exp(2.0 * v) + 1.0)


def _mesh():
  return plsc.VectorSubcoreMesh(core_axis_name="c", subcore_axis_name="s")

_PARAMS = dict(
    compiler_params=pltpu.CompilerParams(use_tc_tiling_on_sc=True,
                                         needs_layout_passes=False))


def _tc_tanh_body(x_ref, o_ref):
  o_ref[...] = jnp.tanh(x_ref[...]).T


def _tc_tanh_rows(embT):
  BCOLS = 4096
  grid = (NSYM + BCOLS - 1) // BCOLS
  return pl.pallas_call(
      _tc_tanh_body,
      grid=(grid,),
      in_specs=[pl.BlockSpec((HIDDEN, BCOLS), lambda i: (0, i))],
      out_specs=pl.BlockSpec((BCOLS, HIDDEN), lambda i: (i, 0)),
      out_shape=jax.ShapeDtypeStruct((NSYM, HIDDEN), jnp.float32),
  )(embT)


def _sc_gather_combine(tt, old_token, new_token, pos, h3, lin, g, b, lg):
  @functools.partial(
      pl.kernel,
      out_type=jax.ShapeDtypeStruct((BATCH // 8, 8, HIDDEN), jnp.float32),
      mesh=_mesh(),
      scratch_types=[
          pltpu.VMEM((B_PER_W + 16,), jnp.int32),
          pltpu.VMEM((B_PER_W + 16,), jnp.int32),
          pltpu.VMEM((B_PER_W + 16,), jnp.int32),
          pltpu.VMEM((HIDDEN,), jnp.float32),
          pltpu.VMEM((HIDDEN,), jnp.float32),
          pltpu.VMEM((HIDDEN,), jnp.float32),
          pltpu.VMEM((HIDDEN,), jnp.float32),
          pltpu.VMEM((GCHUNK, HIDDEN), jnp.float32),
          pltpu.VMEM((GCHUNK, HIDDEN), jnp.float32),
          pltpu.VMEM((TPC, 8, HIDDEN), jnp.float32),
          pltpu.VMEM((TPC, 8, HIDDEN), jnp.float32),
          pltpu.SemaphoreType.DMA,
      ],
      **_PARAMS,
  )
  def k(table, old_hbm, new_hbm, pos_hbm, h_hbm, lin_hbm, g_hbm, b_hbm,
        lg_hbm, out_hbm, oidx, nidx, posv, gv, linv, bv, lgv, xt, yt,
        hb, ob, sem):
    wid = lax.axis_index("s") * 2 + lax.axis_index("c")
    base = wid * B_PER_W
    pltpu.sync_copy(old_hbm.at[pl.ds(base, B_PER_W)],
                    oidx.at[pl.ds(0, B_PER_W)])
    pltpu.sync_copy(new_hbm.at[pl.ds(base, B_PER_W)],
                    nidx.at[pl.ds(0, B_PER_W)])
    pltpu.sync_copy(pos_hbm.at[pl.ds(base, B_PER_W)],
                    posv.at[pl.ds(0, B_PER_W)])
    pltpu.sync_copy(g_hbm, gv)
    pltpu.sync_copy(lin_hbm, linv)
    pltpu.sync_copy(b_hbm, bv)
    pltpu.sync_copy(lg_hbm, lgv)

    gvec = [gv[pl.ds(16 * j, 16)] for j in range(4)]
    linvec = [linv[pl.ds(16 * j, 16)] for j in range(4)]
    bvec = [bv[pl.ds(16 * j, 16)] for j in range(4)]
    lgvec = [lgv[pl.ds(16 * j, 16)] for j in range(4)]

    for c in range(NGCH):
      cb = c * GCHUNK

      def issue(i, _):
        gi = cb + i
        orow = oidx[pl.ds(gi, 16)][0]
        nrow = nidx[pl.ds(gi, 16)][0]
        pltpu.async_copy(table.at[pl.ds(orow, 1), :],
                         xt.at[pl.ds(i, 1), :], sem)
        pltpu.async_copy(table.at[pl.ds(nrow, 1), :],
                         yt.at[pl.ds(i, 1), :], sem)
        return 0
      lax.fori_loop(0, GCHUNK, issue, 0)
      pltpu.sync_copy(h_hbm.at[pl.ds(base // 8 + c * TPC, TPC)], hb)
      pltpu.make_async_copy(table.at[pl.ds(0, GCHUNK), :], xt, sem).wait()
      pltpu.make_async_copy(table.at[pl.ds(0, GCHUNK), :], yt, sem).wait()

      def body(i, _):
        gi = cb + i
        pf = posv[pl.ds(gi, 16)][0].astype(jnp.float32)
        it = lax.shift_right_logical(i, 3)
        is_ = lax.bitwise_and(i, 7)
        for j in range(4):
          sl = pl.ds(16 * j, 16)
          xv = xt[i, sl]
          yv = yt[i, sl]
          hv = hb[it, is_, sl]
          scale = gvec[j] * jnp.exp(pf * lgvec[j]) + pf * linvec[j] + bvec[j]
          ob[it, is_, sl] = hv + (yv - xv) * scale
        return 0
      lax.fori_loop(0, GCHUNK, body, 0)

      pltpu.sync_copy(ob, out_hbm.at[pl.ds(base // 8 + c * TPC, TPC)])

  return k(tt, old_token, new_token, pos, h3, lin, g, b, lg)


def kernel(h, old_token, new_token, pos, embedding, gamma, lin, g, b):
  tt = _tc_tanh_rows(embedding.T)
  h3 = h.reshape(BATCH // 8, 8, HIDDEN)
  lg = jnp.log(gamma)
  out3 = _sc_gather_combine(tt, old_token, new_token, pos, h3,
                            lin, g, b, lg)
  return out3.reshape(BATCH, HIDDEN)


# speedup vs baseline: 5.9466x; 1.0009x over previous
import functools

import jax
import jax.numpy as jnp
from jax import lax
from jax.experimental import pallas as pl
from jax.experimental.pallas import tpu as pltpu
from jax.experimental.pallas import tpu_sc as plsc

HIDDEN = 64
BATCH = 16384
NUM_WORKERS = 32
B_PER_W = BATCH // NUM_WORKERS
CHUNK = 32
NCHUNK = B_PER_W // CHUNK
TPC = CHUNK // 8


def _tanh(v):
  return 1.0 - 2.0 / (jnp.exp(2.0 * v) + 1.0)


def _sc_fused(emb3, old_token, new_token, pos, h3, lin, g, b, lg):
  mesh = plsc.VectorSubcoreMesh(core_axis_name="c", subcore_axis_name="s")

  @functools.partial(
      pl.kernel,
      out_type=jax.ShapeDtypeStruct((BATCH // 8, 8, HIDDEN), jnp.float32),
      mesh=mesh,
      scratch_types=[
          pltpu.VMEM((B_PER_W + 16,), jnp.int32),
          pltpu.VMEM((B_PER_W + 16,), jnp.int32),
          pltpu.VMEM((B_PER_W + 16,), jnp.int32),
          pltpu.VMEM((HIDDEN,), jnp.float32),
          pltpu.VMEM((HIDDEN,), jnp.float32),
          pltpu.VMEM((HIDDEN,), jnp.float32),
          pltpu.VMEM((HIDDEN,), jnp.float32),
          pltpu.VMEM((CHUNK, HIDDEN), jnp.float32),
          pltpu.VMEM((CHUNK, HIDDEN), jnp.float32),
          pltpu.VMEM((TPC, 8, HIDDEN), jnp.float32),
          pltpu.VMEM((TPC, 8, HIDDEN), jnp.float32),
          pltpu.SemaphoreType.DMA,
      ],
      compiler_params=pltpu.CompilerParams(use_tc_tiling_on_sc=True),
  )
  def k(table, old_hbm, new_hbm, pos_hbm, h_hbm, lin_hbm, g_hbm, b_hbm,
        lg_hbm, out_hbm, oidx, nidx, posv,
        gv, linv, bv, lgv, xt, yt, hb, ob, sem):
    wid = lax.axis_index("s") * 2 + lax.axis_index("c")
    base = wid * B_PER_W
    pltpu.sync_copy(old_hbm.at[pl.ds(base, B_PER_W)], oidx.at[pl.ds(0, B_PER_W)])
    pltpu.sync_copy(new_hbm.at[pl.ds(base, B_PER_W)], nidx.at[pl.ds(0, B_PER_W)])
    pltpu.sync_copy(pos_hbm.at[pl.ds(base, B_PER_W)], posv.at[pl.ds(0, B_PER_W)])
    pltpu.sync_copy(g_hbm, gv)
    pltpu.sync_copy(lin_hbm, linv)
    pltpu.sync_copy(b_hbm, bv)
    pltpu.sync_copy(lg_hbm, lgv)

    gvec = [gv[pl.ds(16 * j, 16)] for j in range(4)]
    linvec = [linv[pl.ds(16 * j, 16)] for j in range(4)]
    bvec = [bv[pl.ds(16 * j, 16)] for j in range(4)]
    lgvec = [lgv[pl.ds(16 * j, 16)] for j in range(4)]

    for c in range(NCHUNK):
      cb = c * CHUNK

      def issue(i, _):
        gi = cb + i
        orow = oidx[pl.ds(gi, 16)][0]
        nrow = nidx[pl.ds(gi, 16)][0]
        pltpu.async_copy(table.at[pl.ds(orow, 1), :], xt.at[pl.ds(i, 1), :],
                         sem)
        pltpu.async_copy(table.at[pl.ds(nrow, 1), :], yt.at[pl.ds(i, 1), :],
                         sem)
        return 0
      lax.fori_loop(0, CHUNK, issue, 0)
      pltpu.sync_copy(h_hbm.at[pl.ds(base // 8 + c * TPC, TPC)], hb)
      pltpu.make_async_copy(table.at[pl.ds(0, CHUNK), :], xt, sem).wait()
      pltpu.make_async_copy(table.at[pl.ds(0, CHUNK), :], yt, sem).wait()

      def body(i, _):
        gi = cb + i
        pf = posv[pl.ds(gi, 16)][0].astype(jnp.float32)
        it = lax.shift_right_logical(i, 3)
        is_ = lax.bitwise_and(i, 7)
        for j in range(4):
          sl = pl.ds(16 * j, 16)
          xv = xt[i, sl]
          yv = yt[i, sl]
          hv = hb[it, is_, sl]
          scale = gvec[j] * jnp.exp(pf * lgvec[j]) + pf * linvec[j] + bvec[j]
          ob[it, is_, sl] = hv + (_tanh(yv) - _tanh(xv)) * scale
        return 0
      lax.fori_loop(0, CHUNK, body, 0)

      pltpu.sync_copy(ob, out_hbm.at[pl.ds(base // 8 + c * TPC, TPC)])

  return k(emb3, old_token, new_token, pos, h3, lin, g, b, lg)


def kernel(h, old_token, new_token, pos, embedding, gamma, lin, g, b):
  h3 = h.reshape(BATCH // 8, 8, HIDDEN)
  lg = jnp.log(gamma)
  out3 = _sc_fused(embedding, old_token, new_token, pos, h3, lin, g, b, lg)
  return out3.reshape(BATCH, HIDDEN)
